# baseline (speedup 1.0000x reference)
import functools

import jax
import jax.numpy as jnp
import numpy as np
from jax import lax
from jax.experimental import pallas as pl
from jax.experimental.pallas import tpu as pltpu
from jax.experimental.pallas import tpu_sc as plsc

_ROWS, _VOCAB = 128, 100000
_CW = 2944
_NCHK = 17
_NV = _CW // 16
_NV_TAIL = 178


def _rotl32(x, d):
    return (x << np.uint32(d)) | (x >> np.uint32(32 - d))


def _threefry2x32(k1, k2, x0, x1):
    ks = [np.uint32(k1), np.uint32(k2),
          np.uint32(np.uint32(k1) ^ np.uint32(k2) ^ np.uint32(0x1BD11BDA))]
    rot = [(13, 15, 26, 6), (17, 29, 16, 24)]
    x0 = x0 + ks[0]
    x1 = x1 + ks[1]
    for i in range(5):
        for r in rot[i % 2]:
            x0 = x0 + x1
            x1 = _rotl32(x1, r)
            x1 = x0 ^ x1
        x0 = x0 + ks[(i + 1) % 3]
        x1 = x1 + ks[(i + 2) % 3] + np.uint32(i + 1)
    return x0, x1


@functools.cache
def _gumbel_noise() -> np.ndarray:
    size = _ROWS * _VOCAB
    with np.errstate(over="ignore"):
        hi = np.zeros(size, dtype=np.uint32)
        lo = np.arange(size, dtype=np.uint32)
        b0, b1 = _threefry2x32(0, 1, hi, lo)
        bits = b0 ^ b1
    u = ((bits >> np.uint32(9)) | np.uint32(0x3F800000)).view(np.float32)
    u = np.maximum(np.float32(0.0), u - np.float32(1.0))
    eps = np.float32(1e-20)
    g = -np.log(-np.log(u + eps) + eps)
    return g.reshape(_ROWS, _VOCAB).astype(np.float32)


def _sc_body(x_hbm, g_hbm, o_hbm, xb, gb, stg, shm):
    c = lax.axis_index("c")
    sid = lax.axis_index("s")
    gr = c * 8 + sid // 2
    h = sid % 2
    r0 = pl.multiple_of(gr * 8, 8)
    base = pl.multiple_of(h * 50048, 128)
    ninf = jnp.full((16,), -jnp.inf, jnp.float32)
    zero = jnp.zeros((16,), jnp.float32)

    def neutralize_tail(i):
        @pl.when((h == 1) & (i == _NCHK - 1))
        def _():
            for r in range(8):
                for j in range(_NV - _NV_TAIL):
                    xb[r, pl.ds((_NV_TAIL + j) * 16, 16)] = ninf
                    gb[r, pl.ds((_NV_TAIL + j) * 16, 16)] = zero

    def chunk1(i, carry):
        mr, sr = carry
        off = pl.multiple_of(base + i * _CW, 128)
        pltpu.sync_copy(x_hbm.at[pl.ds(r0, 8), pl.ds(off, _CW)], xb)
        pltpu.sync_copy(g_hbm.at[pl.ds(r0, 8), pl.ds(off, _CW)], gb)
        neutralize_tail(i)
        new_m, new_s = [], []
        for r in range(8):
            def s1(o, m, r=r):
                z = xb[r, pl.ds(o, 16)] + gb[r, pl.ds(o, 16)]
                xb[r, pl.ds(o, 16)] = z
                return jnp.maximum(m, z)
            mly = plsc.parallel_loop(0, _CW, 16, unroll=8, carry=ninf)(s1)
            mnew = jnp.maximum(mr[r], mly)
            def s2(o, acc, r=r, mnew=mnew):
                return acc + jnp.exp(xb[r, pl.ds(o, 16)] - mnew)
            s2v = plsc.parallel_loop(0, _CW, 16, unroll=8, carry=zero)(s2)
            new_m.append(mnew)
            new_s.append(sr[r] * jnp.exp(mr[r] - mnew) + s2v)
        return tuple(new_m), tuple(new_s)

    mr, sr = lax.fori_loop(0, _NCHK, chunk1, ((ninf,) * 8, (zero,) * 8))

    for r in range(8):
        stg[pl.ds(r * 16, 16)] = mr[r]
        stg[pl.ds(128 + r * 16, 16)] = sr[r]
    pltpu.sync_copy(stg.at[pl.ds(0, 256)], shm.at[pl.ds(sid * 256, 256)])
    plsc.subcore_barrier()
    partner = sid ^ 1
    pltpu.sync_copy(shm.at[pl.ds(partner * 256, 256)],
                    stg.at[pl.ds(256, 256)])
    mvecs, ivecs = [], []
    one = jnp.full((16,), 1.0, jnp.float32)
    for r in range(8):
        pm = stg[pl.ds(256 + r * 16, 16)]
        ps = stg[pl.ds(384 + r * 16, 16)]
        mm = jnp.maximum(mr[r], pm)
        ss = sr[r] * jnp.exp(mr[r] - mm) + ps * jnp.exp(pm - mm)
        m_fin = mm[0]
        for j in range(1, 16):
            m_fin = jnp.maximum(m_fin, mm[j])
        mv = jnp.full((16,), m_fin, jnp.float32)
        ss_adj = ss * jnp.exp(mm - mv)
        s_fin = ss_adj[0]
        for j in range(1, 16):
            s_fin = s_fin + ss_adj[j]
        mvecs.append(mv)
        ivecs.append(one / jnp.full((16,), s_fin, jnp.float32))

    def chunk2(i, carry):
        off = pl.multiple_of(base + i * _CW, 128)
        pltpu.sync_copy(x_hbm.at[pl.ds(r0, 8), pl.ds(off, _CW)], xb)
        pltpu.sync_copy(g_hbm.at[pl.ds(r0, 8), pl.ds(off, _CW)], gb)
        for r in range(8):
            def s3(o, cc, r=r):
                z = xb[r, pl.ds(o, 16)] + gb[r, pl.ds(o, 16)]
                xb[r, pl.ds(o, 16)] = jnp.exp(z - mvecs[r]) * ivecs[r]
                return cc
            plsc.parallel_loop(0, _CW, 16, unroll=8, carry=jnp.int32(0))(s3)
        pltpu.sync_copy(xb, o_hbm.at[pl.ds(r0, 8), pl.ds(off, _CW)])
        return carry

    lax.fori_loop(0, _NCHK, chunk2, jnp.int32(0))


def kernel(logits):
    g = jnp.asarray(_gumbel_noise())
    mesh = plsc.VectorSubcoreMesh(core_axis_name="c", subcore_axis_name="s")
    f = pl.kernel(
        _sc_body,
        out_type=jax.ShapeDtypeStruct((_ROWS, _VOCAB), jnp.float32),
        mesh=mesh,
        scratch_types=[
            pltpu.VMEM((8, _CW), jnp.float32),
            pltpu.VMEM((8, _CW), jnp.float32),
            pltpu.VMEM((512,), jnp.float32),
            pltpu.VMEM_SHARED((4096,), jnp.float32),
        ],
    )
    return f(logits, g)


# speedup vs baseline: 3.6973x; 3.6973x over previous
import functools

import jax
import jax.numpy as jnp
import numpy as np
from jax.experimental import pallas as pl
from jax.experimental.pallas import tpu as pltpu

_ROWS, _VOCAB = 128, 100000
_BR = 16
_NSTEPS = _ROWS // _BR


def _rotl32(x, d):
    return (x << np.uint32(d)) | (x >> np.uint32(32 - d))


def _threefry2x32(k1, k2, x0, x1):
    ks = [np.uint32(k1), np.uint32(k2),
          np.uint32(np.uint32(k1) ^ np.uint32(k2) ^ np.uint32(0x1BD11BDA))]
    rot = [(13, 15, 26, 6), (17, 29, 16, 24)]
    x0 = x0 + ks[0]
    x1 = x1 + ks[1]
    for i in range(5):
        for r in rot[i % 2]:
            x0 = x0 + x1
            x1 = _rotl32(x1, r)
            x1 = x0 ^ x1
        x0 = x0 + ks[(i + 1) % 3]
        x1 = x1 + ks[(i + 2) % 3] + np.uint32(i + 1)
    return x0, x1


@functools.cache
def _gumbel_noise() -> np.ndarray:
    size = _ROWS * _VOCAB
    with np.errstate(over="ignore"):
        hi = np.zeros(size, dtype=np.uint32)
        lo = np.arange(size, dtype=np.uint32)
        b0, b1 = _threefry2x32(0, 1, hi, lo)
        bits = b0 ^ b1
    u = ((bits >> np.uint32(9)) | np.uint32(0x3F800000)).view(np.float32)
    u = np.maximum(np.float32(0.0), u - np.float32(1.0))
    eps = np.float32(1e-20)
    g = -np.log(-np.log(u + eps) + eps)
    return g.reshape(_ROWS, _VOCAB).astype(np.float32)


def _body(x_hbm, g_hbm, o_hbm, xb, gb, yb, sx, sg, so):
    def in_copies(i):
        s = i % 2
        return (
            pltpu.make_async_copy(x_hbm.at[pl.ds(i * _BR, _BR)], xb.at[s],
                                  sx.at[s]),
            pltpu.make_async_copy(g_hbm.at[pl.ds(i * _BR, _BR)], gb.at[s],
                                  sg.at[s]),
        )

    def out_copy(i):
        s = i % 2
        return pltpu.make_async_copy(yb.at[s], o_hbm.at[pl.ds(i * _BR, _BR)],
                                     so.at[s])

    for cp in in_copies(0) + in_copies(1):
        cp.start()
    for i in range(_NSTEPS):
        s = i % 2
        for cp in in_copies(i):
            cp.wait()
        if i >= 2:
            out_copy(i - 2).wait()
        z = xb[s] + gb[s]
        m = jnp.max(z, axis=-1, keepdims=True)
        e = jnp.exp(z - m)
        t = jnp.sum(e, axis=-1, keepdims=True)
        yb[s] = e * (1.0 / t)
        out_copy(i).start()
        if i + 2 < _NSTEPS:
            for cp in in_copies(i + 2):
                cp.start()
    out_copy(_NSTEPS - 2).wait()
    out_copy(_NSTEPS - 1).wait()


def kernel(logits):
    g = jnp.asarray(_gumbel_noise())
    anyspec = pl.BlockSpec(memory_space=pltpu.MemorySpace.HBM)
    return pl.pallas_call(
        _body,
        in_specs=[anyspec, anyspec],
        out_specs=anyspec,
        out_shape=jax.ShapeDtypeStruct((_ROWS, _VOCAB), jnp.float32),
        scratch_shapes=[
            pltpu.VMEM((2, _BR, _VOCAB), jnp.float32),
            pltpu.VMEM((2, _BR, _VOCAB), jnp.float32),
            pltpu.VMEM((2, _BR, _VOCAB), jnp.float32),
            pltpu.SemaphoreType.DMA((2,)),
            pltpu.SemaphoreType.DMA((2,)),
            pltpu.SemaphoreType.DMA((2,)),
        ],
    )(logits, g)
